# Initial kernel scaffold; baseline (speedup 1.0000x reference)
import jax
import jax.numpy as jnp
from jax.experimental import pallas as pl


def kernel(features, adj, weight, bias, gamma, beta):
    raise NotImplementedError("write your pallas kernel here")


# speedup vs baseline: 1.0865x; 1.0865x over previous
import functools

import jax
import jax.numpy as jnp
from jax.experimental import pallas as pl
from jax.experimental.pallas import tpu as pltpu

IN_DIM = 128
OUT_DIM = 128
N = 10000
BLK = 400
NBLK = N // BLK


def _gnn_body(feat_ref, w_ref, b_ref, g_ref, be_ref, adj_ref, out_ref,
              support_ref, sum_ref, sq_ref):
    i = pl.program_id(0)

    @pl.when(i == 0)
    def _init():
        support_ref[...] = jnp.dot(
            feat_ref[...], w_ref[...], preferred_element_type=jnp.float32)
        sum_ref[...] = jnp.zeros_like(sum_ref)
        sq_ref[...] = jnp.zeros_like(sq_ref)

    blk = jnp.dot(
        adj_ref[...], support_ref[...], preferred_element_type=jnp.float32)
    blk = blk + b_ref[...]
    out_ref[pl.ds(i * BLK, BLK), :] = blk
    sum_ref[...] += jnp.sum(blk, axis=0, keepdims=True)
    sq_ref[...] += jnp.sum(blk * blk, axis=0, keepdims=True)

    @pl.when(i == NBLK - 1)
    def _finalize():
        mean = sum_ref[...] * (1.0 / N)
        var = sq_ref[...] * (1.0 / N) - mean * mean
        scale = jax.lax.rsqrt(var + 1e-5) * g_ref[...]
        shift = be_ref[...] - mean * scale
        out_ref[...] = jnp.maximum(out_ref[...] * scale + shift, 0.0)


@jax.jit
def kernel(features, adj, weight, bias, gamma, beta):
    grid_spec = pltpu.PrefetchScalarGridSpec(
        num_scalar_prefetch=0,
        grid=(NBLK,),
        in_specs=[
            pl.BlockSpec((N, IN_DIM), lambda i: (0, 0)),
            pl.BlockSpec((IN_DIM, OUT_DIM), lambda i: (0, 0)),
            pl.BlockSpec((1, OUT_DIM), lambda i: (0, 0)),
            pl.BlockSpec((1, OUT_DIM), lambda i: (0, 0)),
            pl.BlockSpec((1, OUT_DIM), lambda i: (0, 0)),
            pl.BlockSpec((BLK, N), lambda i: (i, 0)),
        ],
        out_specs=pl.BlockSpec((N, OUT_DIM), lambda i: (0, 0)),
        scratch_shapes=[
            pltpu.VMEM((N, OUT_DIM), jnp.float32),
            pltpu.VMEM((1, OUT_DIM), jnp.float32),
            pltpu.VMEM((1, OUT_DIM), jnp.float32),
        ],
    )
    return pl.pallas_call(
        _gnn_body,
        grid_spec=grid_spec,
        out_shape=jax.ShapeDtypeStruct((N, OUT_DIM), jnp.float32),
        compiler_params=pltpu.CompilerParams(
            dimension_semantics=("arbitrary",),
        ),
    )(features, weight, bias.reshape(1, OUT_DIM), gamma.reshape(1, OUT_DIM),
      beta.reshape(1, OUT_DIM), adj)
